# Initial kernel scaffold; baseline (speedup 1.0000x reference)
import jax
import jax.numpy as jnp
from jax.experimental import pallas as pl


def kernel(x, table):
    raise NotImplementedError("write your pallas kernel here")


# speedup vs baseline: 1.2835x; 1.2835x over previous
import functools

import jax
import jax.numpy as jnp
from jax import lax
from jax.experimental import pallas as pl
from jax.experimental.pallas import tpu as pltpu
from jax.experimental.pallas import tpu_sc as plsc

VOCAB = 1000000
EMBED = 32
BATCH = 16384
HIST = 50

NC = 2
NS = 16
NW = NC * NS

ROWS = BATCH * HIST // 128
ROWS_PER_W = ROWS // NW
CK = 8
N_CHUNK = ROWS_PER_W // CK


def _make_kernel():
  mesh = plsc.VectorSubcoreMesh(core_axis_name="c", subcore_axis_name="s")

  @functools.partial(
      pl.kernel,
      out_type=jax.ShapeDtypeStruct((ROWS, 128, EMBED), jnp.float32),
      mesh=mesh,
      compiler_params=pltpu.CompilerParams(use_tc_tiling_on_sc=False),
      scratch_types=[
          pltpu.VMEM((CK, 128), jnp.int32),
          pltpu.VMEM((CK, 128, EMBED), jnp.float32),
          pltpu.SemaphoreType.DMA,
      ],
  )
  def gather_kernel(idx_hbm, table_hbm, out_hbm, idx_v, rows_v, sem):
    wid = lax.axis_index("s") * NC + lax.axis_index("c")
    base = wid * ROWS_PER_W

    @pl.loop(0, N_CHUNK)
    def _chunk(c):
      r0 = base + c * CK
      pltpu.sync_copy(idx_hbm.at[pl.ds(r0, CK)], idx_v)
      for j in range(CK):
        pltpu.async_copy(table_hbm.at[idx_v.at[j]], rows_v.at[j], sem)
      for j in range(CK):
        pltpu.make_async_copy(table_hbm.at[idx_v.at[j]], rows_v.at[j], sem).wait()
      pltpu.sync_copy(rows_v, out_hbm.at[pl.ds(r0, CK)])

  return gather_kernel


_gather = _make_kernel()


@jax.jit
def kernel(x, table):
  idx = x.reshape(ROWS, 128)
  out = _gather(idx, table)
  return out.reshape(BATCH, HIST, EMBED)


# speedup vs baseline: 1.7409x; 1.3563x over previous
import functools

import jax
import jax.numpy as jnp
from jax import lax
from jax.experimental import pallas as pl
from jax.experimental.pallas import tpu as pltpu
from jax.experimental.pallas import tpu_sc as plsc

VOCAB = 1000000
EMBED = 32
BATCH = 16384
HIST = 50

NC = 2
NS = 16
NW = NC * NS

B_PER_W = BATCH // NW
GK = 16
N_CHUNK = B_PER_W // GK


def _make_kernel():
  mesh = plsc.VectorSubcoreMesh(core_axis_name="c", subcore_axis_name="s")

  @functools.partial(
      pl.kernel,
      out_type=jax.ShapeDtypeStruct((BATCH, HIST, EMBED), jnp.float32),
      mesh=mesh,
      compiler_params=pltpu.CompilerParams(use_tc_tiling_on_sc=False),
      scratch_types=[
          pltpu.VMEM((GK, HIST), jnp.int32),
          pltpu.VMEM((GK, HIST, EMBED), jnp.float32),
          pltpu.SemaphoreType.DMA,
      ],
  )
  def gather_kernel(x_hbm, table_hbm, out_hbm, idx_v, rows_v, sem):
    wid = lax.axis_index("s") * NC + lax.axis_index("c")
    base = wid * B_PER_W

    @pl.loop(0, N_CHUNK)
    def _chunk(c):
      b0 = base + c * GK
      pltpu.sync_copy(x_hbm.at[pl.ds(b0, GK)], idx_v)
      for t in range(GK):
        pltpu.async_copy(table_hbm.at[idx_v.at[t]], rows_v.at[t], sem)
      for t in range(GK):
        pltpu.make_async_copy(table_hbm.at[idx_v.at[t]], rows_v.at[t], sem).wait()
      pltpu.sync_copy(rows_v, out_hbm.at[pl.ds(b0, GK)])

  return gather_kernel


_gather = _make_kernel()


@jax.jit
def kernel(x, table):
  return _gather(x, table)
